# baseline (speedup 1.0000x reference)
import functools

import jax
import jax.numpy as jnp
import numpy as np
from jax import lax
from jax.experimental import pallas as pl
from jax.experimental.pallas import tpu as pltpu
from jax.experimental.pallas import tpu_sc as plsc

_OFF_W1 = 16
_OFF_B1 = 4112
_OFF_W2 = 4128
_OFF_B2 = 8224
_NPARAMS = 8480

_CH_PER_TILE = 80
_ROWS_PER_W = 1024


def _sc_kernel_call(xf, params, idx1d):
    mesh = plsc.VectorSubcoreMesh(core_axis_name="c", subcore_axis_name="s")
    nc = plsc.get_sparse_core_info().num_cores

    @functools.partial(
        pl.kernel,
        mesh=mesh,
        compiler_params=pltpu.CompilerParams(needs_layout_passes=False),
        out_type=jax.ShapeDtypeStruct((32, 9, _ROWS_PER_W), jnp.float32),
        scratch_types=[
            pltpu.VMEM((_CH_PER_TILE * 625,), jnp.float32),
            pltpu.VMEM((_NPARAMS,), jnp.float32),
            pltpu.VMEM((_CH_PER_TILE * 16,), jnp.float32),
            pltpu.VMEM((1280 * 16,), jnp.float32),
            pltpu.VMEM((256,), jnp.float32),
            pltpu.VMEM((1280,), jnp.float32),
            pltpu.VMEM((_CH_PER_TILE * 16,), jnp.float32),
            pltpu.VMEM((1280 * 16,), jnp.float32),
            pltpu.VMEM((_ROWS_PER_W,), jnp.int32),
            pltpu.VMEM((9, _ROWS_PER_W), jnp.float32),
            pltpu.VMEM_SHARED((1280 * 16,), jnp.float32),
            pltpu.VMEM_SHARED((1280 * 16,), jnp.float32),
        ],
    )
    def k(xf_hbm, params_hbm, idx_hbm, out_hbm,
          xbuf, pbuf, msplat, means_v, xabuf, attn_v, tbuf, table_v,
          idx_v, obuf, means_sh, table_sh):
        s = lax.axis_index("s")
        wid = s * nc + lax.axis_index("c")
        iota = lax.iota(jnp.int32, 16)

        pltpu.sync_copy(xf_hbm.at[pl.ds(s * (_CH_PER_TILE * 625),
                                        _CH_PER_TILE * 625)], xbuf)
        pltpu.sync_copy(params_hbm, pbuf)
        pltpu.sync_copy(idx_hbm.at[pl.ds(wid * _ROWS_PER_W, _ROWS_PER_W)],
                        idx_v)

        def mean_body(kk, carry):
            base = kk * 625 + iota
            acc = jnp.zeros((16,), jnp.float32)
            for j in range(39):
                acc = acc + plsc.load_gather(xbuf, [base + 16 * j])
            tail = plsc.load_gather(xbuf, [base + 609])
            acc = acc + jnp.where(iota == 15, tail, 0.0)
            m = jnp.sum(acc) * (1.0 / 625.0)
            msplat[pl.ds(kk * 16, 16)] = jnp.full((16,), m, jnp.float32)
            return carry

        lax.fori_loop(0, _CH_PER_TILE, mean_body, 0)
        pltpu.sync_copy(msplat, means_sh.at[pl.ds(s * (_CH_PER_TILE * 16),
                                                  _CH_PER_TILE * 16)])
        plsc.subcore_barrier()
        pltpu.sync_copy(means_sh, means_v)

        taps = pbuf[pl.ds(0, 16)]
        b1v = pbuf[pl.ds(_OFF_B1, 16)]

        for b in range(5):
            def conv_body(g, carry, b=b):
                addr0 = (b * 256 + g * 16) * 16 + iota * 16
                acc = plsc.load_gather(means_v, [addr0]) + taps[7]
                for t in range(7):
                    d = t - 3
                    cd = g * 16 + iota + d
                    valid = (cd >= 0) & (cd <= 255)
                    cdc = jnp.clip(cd, 0, 255)
                    mv = plsc.load_gather(means_v, [(b * 256 + cdc) * 16])
                    acc = acc + taps[t] * jnp.where(valid, mv, 0.0)
                xabuf[pl.ds(g * 16, 16)] = acc
                return carry

            lax.fori_loop(0, 16, conv_body, 0)

            def fc1_body(g, h):
                xg = xabuf[pl.ds(g * 16, 16)]
                for l in range(16):
                    h = h + xg[l] * pbuf[pl.ds(_OFF_W1 + g * 256 + l * 16, 16)]
                return h

            h = lax.fori_loop(0, 16, fc1_body, b1v)
            h = jnp.maximum(h, 0.0)

            def fc2_body(g, carry, b=b, h=h):
                lv = pbuf[pl.ds(_OFF_B2 + g * 16, 16)]
                for j in range(16):
                    lv = lv + h[j] * pbuf[pl.ds(_OFF_W2 + j * 256 + g * 16, 16)]
                attn = 1.0 / (1.0 + jnp.exp(-lv))
                attn_v[pl.ds(b * 256 + g * 16, 16)] = attn
                return carry

            lax.fori_loop(0, 16, fc2_body, 0)

        q3 = iota // 3
        pcol = jnp.where(iota < 9, 286 + q3 * 25 + (iota - q3 * 3), 0)
        mask9 = jnp.where(iota < 9, 1.0, 0.0).astype(jnp.float32)
        for kk in range(5):
            avg = attn_v[pl.ds(s * _CH_PER_TILE + kk * 16, 16)]
            for l in range(16):
                krow = kk * 16 + l
                pv = plsc.load_gather(xbuf, [krow * 625 + pcol])
                tbuf[pl.ds(krow * 16, 16)] = pv * avg[l] * mask9
        pltpu.sync_copy(tbuf, table_sh.at[pl.ds(s * (_CH_PER_TILE * 16),
                                                _CH_PER_TILE * 16)])
        plsc.subcore_barrier()
        pltpu.sync_copy(table_sh, table_v)

        def gather_body(i, carry):
            addr = idx_v[pl.ds(i * 16, 16)] * 16
            for cc in range(9):
                v = plsc.load_gather(table_v, [addr + cc])
                obuf[cc, pl.ds(i * 16, 16)] = v
            return carry

        lax.fori_loop(0, _ROWS_PER_W // 16, gather_body, 0)
        pltpu.sync_copy(obuf, out_hbm.at[wid])

    return k(xf, params, idx1d)


def _gather_indices(B, C):
    np.random.seed(0)
    idx = np.empty((B, 5, 100), np.int64)
    for i in range(B):
        for k in range(5):
            idx[i, k] = np.random.randint(0, C, 100)
    g = np.empty((B, 5, 100), np.int64)
    for m in range(5):
        g[:, m, :] = (4 - m) * C + idx[:, 4 - m, :]
    return g.reshape(-1)


def kernel(x, conv1_w, conv1_b, conv2_w, conv2_b, conv3_w, conv3_b,
           fc1_w, fc1_b, fc2_w, fc2_b):
    B, C, H, W = x.shape
    xf = x[:5].reshape(5 * C * H * W)

    w3 = conv1_w[0, 0, 1, :]
    w5 = conv2_w[0, 0, 2, :]
    w7 = conv3_w[0, 0, 3, :]
    taps = w7 + jnp.pad(w5, (1, 1)) + jnp.pad(w3, (2, 2))
    bsum = conv1_b[0] + conv2_b[0] + conv3_b[0]
    params = jnp.concatenate([
        taps, bsum[None], jnp.zeros(8, jnp.float32),
        fc1_w.reshape(C // 16, C).T.reshape(-1),
        fc1_b,
        fc2_w.reshape(C, C // 16).T.reshape(-1),
        fc2_b,
    ])

    n_workers = 32
    total = n_workers * _ROWS_PER_W
    gflat = np.zeros(total, np.int32)
    gflat[:B * 500] = _gather_indices(B, C)

    cols = _sc_kernel_call(xf, params, jnp.asarray(gflat))
    rows = jnp.transpose(cols, (0, 2, 1)).reshape(total, 9)
    return rows[:B * 500].reshape(B, 5, 100, 3, 3)
